# baseline (speedup 1.0000x reference)
import jax
import jax.numpy as jnp
from jax import lax
from jax.experimental import pallas as pl
from jax.experimental.pallas import tpu as pltpu
from jax.experimental.pallas import tpu_sc as plsc

LANES = 16
NC = 2
NS = 16
NW = NC * NS
EB = 80


def _mm2_body(x_ref, w1_ref, w2_ref, u_ref, v_ref):
    xb = x_ref[...]
    u_ref[...] = jnp.dot(xb, w1_ref[...], preferred_element_type=jnp.float32)
    v_ref[...] = jnp.dot(xb, w2_ref[...], preferred_element_type=jnp.float32)


def _dual_matmul(x, w1, w2, rb=1000):
    n, d = x.shape
    l = w1.shape[1]
    return pl.pallas_call(
        _mm2_body,
        grid=(n // rb,),
        in_specs=[pl.BlockSpec((rb, d), lambda i: (i, 0)),
                  pl.BlockSpec((d, l), lambda i: (0, 0)),
                  pl.BlockSpec((d, l), lambda i: (0, 0))],
        out_specs=[pl.BlockSpec((rb, l), lambda i: (i, 0)),
                   pl.BlockSpec((rb, l), lambda i: (i, 0))],
        out_shape=[jax.ShapeDtypeStruct((n, l), jnp.float32),
                   jax.ShapeDtypeStruct((n, l), jnp.float32)],
    )(x, w1, w2)


def _sreduce_body(s_ref, r_ref):
    s = jnp.sum(s_ref[...], axis=0)
    r_ref[...] = (1.0 / (s + 1e-16))[:, None]


def _sreduce(s_mat):
    nw, np_ = s_mat.shape
    return pl.pallas_call(
        _sreduce_body,
        grid=(1,),
        in_specs=[pl.BlockSpec((nw, np_), lambda i: (0, 0))],
        out_specs=pl.BlockSpec((np_, 1), lambda i: (0, 0)),
        out_shape=jax.ShapeDtypeStruct((np_, 1), jnp.float32),
    )(s_mat)


def _norm_mm2_body(acc0_ref, acc1_ref, r_ref, bias_ref, w1_ref, w2_ref,
                   u_ref, v_ref):
    acc = acc0_ref[...] + acc1_ref[...]
    o = acc * r_ref[...] + bias_ref[...]
    u_ref[...] = jnp.dot(o, w1_ref[...], preferred_element_type=jnp.float32)
    v_ref[...] = jnp.dot(o, w2_ref[...], preferred_element_type=jnp.float32)


def _norm_mm2(acc0, acc1, rcol, bias, w1, w2, rb=1000):
    n, d = acc0.shape
    l = w1.shape[1]
    nb = n // rb
    u, v = pl.pallas_call(
        _norm_mm2_body,
        grid=(nb,),
        in_specs=[pl.BlockSpec((rb, d), lambda i: (i, 0)),
                  pl.BlockSpec((rb, d), lambda i: (i, 0)),
                  pl.BlockSpec((rb, 1), lambda i: (i, 0)),
                  pl.BlockSpec((1, d), lambda i: (0, 0)),
                  pl.BlockSpec((d, l), lambda i: (0, 0)),
                  pl.BlockSpec((d, l), lambda i: (0, 0))],
        out_specs=[pl.BlockSpec((rb, l), lambda i: (i, 0)),
                   pl.BlockSpec((rb, l), lambda i: (i, 0))],
        out_shape=[jax.ShapeDtypeStruct((n, l), jnp.float32),
                   jax.ShapeDtypeStruct((n, l), jnp.float32)],
    )(acc0, acc1, rcol, bias, w1, w2)
    return u, v


def _final_body(acc0_ref, acc1_ref, r_ref, row_ref, out_ref):
    acc = acc0_ref[...] + acc1_ref[...]
    out_ref[...] = acc * r_ref[...] + row_ref[...]


def _final(acc0, acc1, rcol, row, rb=1000):
    n, d = acc0.shape
    nb = n // rb
    return pl.pallas_call(
        _final_body,
        grid=(nb,),
        in_specs=[pl.BlockSpec((rb, d), lambda i: (i, 0)),
                  pl.BlockSpec((rb, d), lambda i: (i, 0)),
                  pl.BlockSpec((rb, 1), lambda i: (i, 0)),
                  pl.BlockSpec((1, d), lambda i: (0, 0))],
        out_specs=pl.BlockSpec((rb, d), lambda i: (i, 0)),
        out_shape=jax.ShapeDtypeStruct((n, d), jnp.float32),
    )(acc0, acc1, rcol, row)


def _sload(ref1d, i):
    return ref1d[pl.ds(i, LANES)][0]


def _make_edge_kernel(has_edge, e_total, n_nodes, n_pad, d):
    epw = e_total // NW
    nchunks = epw // EB
    assert e_total % NW == 0 and epw % EB == 0 and EB % LANES == 0
    rows_per_tile = n_pad // NS
    assert n_pad % NS == 0 and rows_per_tile % EB == 0
    nzc = rows_per_tile // EB
    groups = EB // LANES
    nch = d // LANES

    def body(*refs):
        n_in = 8 if has_edge else 5
        n_out = 4 if has_edge else 3
        ins, outs, scr = refs[:n_in], refs[n_in:n_in + n_out], refs[n_in + n_out:]
        if has_edge:
            (src_hbm, dst_hbm, u_hbm, v_hbm, att_hbm, wev_hbm, pprev_hbm,
             rprev_hbm) = ins
            p_out, sacc_out, acc_out, alpha_out = outs
            (u_buf, v_buf, src_idx, dst_idx, dst_pad, p_buf, att_buf, s_tile,
             acc_sh, wev_buf, pprev_buf, alpha_buf, rprev_buf) = scr
        else:
            src_hbm, dst_hbm, u_hbm, v_hbm, att_hbm = ins
            p_out, sacc_out, acc_out = outs
            (u_buf, v_buf, src_idx, dst_idx, dst_pad, p_buf, att_buf, s_tile,
             acc_sh) = scr

        c = lax.axis_index("c")
        sx = lax.axis_index("s")
        wid = c * NS + sx
        tile_base = wid * epw
        row0 = sx * rows_per_tile

        pltpu.sync_copy(att_hbm, att_buf)
        if has_edge:
            pltpu.sync_copy(wev_hbm, wev_buf)
            pltpu.sync_copy(rprev_hbm, rprev_buf.at[pl.ds(0, n_nodes)])

        def _zu(i, _):
            for j in range(nch):
                u_buf[i, pl.ds(j * LANES, LANES)] = jnp.zeros((LANES,),
                                                              jnp.float32)
            return 0
        lax.fori_loop(0, EB, _zu, 0)

        def _zs(i, _):
            s_tile[pl.ds(i * LANES, LANES)] = jnp.zeros((LANES,), jnp.float32)
            return 0
        lax.fori_loop(0, (n_pad + LANES) // LANES, _zs, 0)

        for j in range(nzc):
            pltpu.sync_copy(u_buf, acc_sh.at[pl.ds(row0 + j * EB, EB)])
        plsc.subcore_barrier()

        def chunk(ci, _):
            base = tile_base + ci * EB
            pltpu.sync_copy(src_hbm.at[pl.ds(base, EB)], src_idx)
            pltpu.sync_copy(dst_hbm.at[pl.ds(base, EB)], dst_idx)
            pltpu.sync_copy(dst_hbm.at[pl.ds(base, EB)],
                            dst_pad.at[pl.ds(0, EB)])
            pltpu.sync_copy(u_hbm.at[src_idx], u_buf)
            pltpu.sync_copy(v_hbm.at[dst_idx], v_buf)
            if has_edge:
                pltpu.sync_copy(pprev_hbm.at[pl.ds(base, EB)],
                                pprev_buf.at[pl.ds(0, EB)])

            att_c = [att_buf[pl.ds(j * LANES, LANES)] for j in range(nch)]
            if has_edge:
                wev_c = [wev_buf[pl.ds(j * LANES, LANES)] for j in range(nch)]

            for g in range(groups):
                lane = lax.iota(jnp.int32, LANES)

                def edgeA(l, carry):
                    lv, av = carry
                    e = g * LANES + l
                    if has_edge:
                        alpha = _sload(pprev_buf, e) * \
                            _sload(rprev_buf, _sload(dst_pad, e))
                        av = jnp.where(lane == l, alpha, av)
                    acc = jnp.zeros((LANES,), jnp.float32)
                    for j in range(nch):
                        z = u_buf[e, pl.ds(j * LANES, LANES)] + \
                            v_buf[e, pl.ds(j * LANES, LANES)]
                        if has_edge:
                            z = z + alpha * wev_c[j]
                        acc = acc + att_c[j] * jnp.maximum(z, 0.2 * z)
                    lv = jnp.where(lane == l, jnp.sum(acc), lv)
                    return (lv, av)
                z16 = jnp.zeros((LANES,), jnp.float32)
                lv, av = lax.fori_loop(0, LANES, edgeA, (z16, z16),
                                       unroll=4)
                p_buf[pl.ds(g * LANES, LANES)] = jnp.exp(lv)
                if has_edge:
                    alpha_buf[pl.ds(g * LANES, LANES)] = av

            def edgeB(e, _):
                p = _sload(p_buf, e)
                dste = _sload(dst_pad, e)
                win = s_tile[pl.ds(dste, LANES)]
                s_tile[pl.ds(dste, LANES)] = win + jnp.where(
                    lax.iota(jnp.int32, LANES) == 0, p, 0.0)
                for j in range(nch):
                    u_buf[e, pl.ds(j * LANES, LANES)] = \
                        p * u_buf[e, pl.ds(j * LANES, LANES)]
                return 0
            lax.fori_loop(0, EB, edgeB, 0, unroll=4)

            pltpu.sync_copy(p_buf.at[pl.ds(0, EB)], p_out.at[pl.ds(base, EB)])
            if has_edge:
                pltpu.sync_copy(alpha_buf, alpha_out.at[pl.ds(base, EB)])
            pltpu.sync_copy(u_buf, acc_sh.at[dst_idx], add=True)
            return 0

        lax.fori_loop(0, nchunks, chunk, 0)
        plsc.subcore_barrier()

        out_base = c * n_pad + row0
        for j in range(nzc):
            pltpu.sync_copy(acc_sh.at[pl.ds(row0 + j * EB, EB)], u_buf)
            pltpu.sync_copy(u_buf, acc_out.at[pl.ds(out_base + j * EB, EB)])
        pltpu.sync_copy(s_tile.at[pl.ds(0, n_pad)],
                        sacc_out.at[pl.ds(wid * n_pad, n_pad)])

    out_type = [jax.ShapeDtypeStruct((e_total,), jnp.float32),
                jax.ShapeDtypeStruct((NW * n_pad,), jnp.float32),
                jax.ShapeDtypeStruct((NC * n_pad, d), jnp.float32)]
    scratch = [pltpu.VMEM((EB, d), jnp.float32),
               pltpu.VMEM((EB, d), jnp.float32),
               pltpu.VMEM((EB,), jnp.int32),
               pltpu.VMEM((EB,), jnp.int32),
               pltpu.VMEM((EB + LANES,), jnp.int32),
               pltpu.VMEM((EB + LANES,), jnp.float32),
               pltpu.VMEM((d,), jnp.float32),
               pltpu.VMEM((n_pad + LANES,), jnp.float32),
               pltpu.VMEM_SHARED((n_pad, d), jnp.float32)]
    if has_edge:
        out_type.append(jax.ShapeDtypeStruct((e_total,), jnp.float32))
        scratch += [pltpu.VMEM((d,), jnp.float32),
                    pltpu.VMEM((EB + LANES,), jnp.float32),
                    pltpu.VMEM((EB,), jnp.float32),
                    pltpu.VMEM((n_nodes + LANES,), jnp.float32)]

    mesh = plsc.VectorSubcoreMesh(core_axis_name="c", subcore_axis_name="s")
    return pl.kernel(body, out_type=out_type, mesh=mesh, scratch_types=scratch,
                     compiler_params=pltpu.CompilerParams(
                         needs_layout_passes=False))


def kernel(x, edge_index, mmse_score, W_src1, W_dst1, att1, bias1,
           W_src2, W_dst2, att2, W_edge2, bias2, W_mmse, b_mmse):
    n, d = x.shape
    e_total = edge_index.shape[1]
    src = edge_index[0]
    dst = edge_index[1]

    n_pad = 10240 if n == 10000 else ((n + NS * EB - 1) // (NS * EB)) * (NS * EB)

    u1, v1 = _dual_matmul(x, W_src1, W_dst1)

    layer1 = _make_edge_kernel(False, e_total, n, n_pad, d)
    p1, sacc1, acc1 = layer1(src, dst, u1, v1, att1)

    r1 = _sreduce(sacc1.reshape(NW, n_pad))
    u2, v2 = _norm_mm2(acc1[:n], acc1[n_pad:n_pad + n], r1[:n],
                       bias1.reshape(1, d), W_src2, W_dst2)

    layer2 = _make_edge_kernel(True, e_total, n, n_pad, d)
    p2, sacc2, acc2, alpha1 = layer2(src, dst, u2, v2, att2, W_edge2[0],
                                     p1, r1.reshape(n_pad)[:n])

    r2 = _sreduce(sacc2.reshape(NW, n_pad))
    row = mmse_score[0, 0] * W_mmse + b_mmse[None, :] + bias2[None, :]
    gf = _final(acc2[:n], acc2[n_pad:n_pad + n], r2[:n], row.reshape(1, d))
    return (gf, alpha1)


# speedup vs baseline: 1.3349x; 1.3349x over previous
import jax
import jax.numpy as jnp
from jax import lax
from jax.experimental import pallas as pl
from jax.experimental.pallas import tpu as pltpu
from jax.experimental.pallas import tpu_sc as plsc

LANES = 16
NC = 2
NS = 16
NW = NC * NS
EB = 80


def _mm2_body(x_ref, w1_ref, w2_ref, u_ref, v_ref):
    xb = x_ref[...]
    u_ref[...] = jnp.dot(xb, w1_ref[...], preferred_element_type=jnp.float32)
    v_ref[...] = jnp.dot(xb, w2_ref[...], preferred_element_type=jnp.float32)


def _dual_matmul(x, w1, w2, rb=1000):
    n, d = x.shape
    l = w1.shape[1]
    return pl.pallas_call(
        _mm2_body,
        grid=(n // rb,),
        in_specs=[pl.BlockSpec((rb, d), lambda i: (i, 0)),
                  pl.BlockSpec((d, l), lambda i: (0, 0)),
                  pl.BlockSpec((d, l), lambda i: (0, 0))],
        out_specs=[pl.BlockSpec((rb, l), lambda i: (i, 0)),
                   pl.BlockSpec((rb, l), lambda i: (i, 0))],
        out_shape=[jax.ShapeDtypeStruct((n, l), jnp.float32),
                   jax.ShapeDtypeStruct((n, l), jnp.float32)],
    )(x, w1, w2)


def _sreduce_body(s_ref, r_ref):
    s = jnp.sum(s_ref[...], axis=0)
    r_ref[...] = (1.0 / (s + 1e-16))[:, None]


def _sreduce(s_mat):
    nw, np_ = s_mat.shape
    return pl.pallas_call(
        _sreduce_body,
        grid=(1,),
        in_specs=[pl.BlockSpec((nw, np_), lambda i: (0, 0))],
        out_specs=pl.BlockSpec((np_, 1), lambda i: (0, 0)),
        out_shape=jax.ShapeDtypeStruct((np_, 1), jnp.float32),
    )(s_mat)


def _norm_mm2_body(acc0_ref, acc1_ref, r_ref, bias_ref, w1_ref, w2_ref,
                   u_ref, v_ref):
    acc = acc0_ref[...] + acc1_ref[...]
    o = acc * r_ref[...] + bias_ref[...]
    u_ref[...] = jnp.dot(o, w1_ref[...], preferred_element_type=jnp.float32)
    v_ref[...] = jnp.dot(o, w2_ref[...], preferred_element_type=jnp.float32)


def _norm_mm2(acc0, acc1, rcol, bias, w1, w2, rb=1000):
    n, d = acc0.shape
    l = w1.shape[1]
    nb = n // rb
    u, v = pl.pallas_call(
        _norm_mm2_body,
        grid=(nb,),
        in_specs=[pl.BlockSpec((rb, d), lambda i: (i, 0)),
                  pl.BlockSpec((rb, d), lambda i: (i, 0)),
                  pl.BlockSpec((rb, 1), lambda i: (i, 0)),
                  pl.BlockSpec((1, d), lambda i: (0, 0)),
                  pl.BlockSpec((d, l), lambda i: (0, 0)),
                  pl.BlockSpec((d, l), lambda i: (0, 0))],
        out_specs=[pl.BlockSpec((rb, l), lambda i: (i, 0)),
                   pl.BlockSpec((rb, l), lambda i: (i, 0))],
        out_shape=[jax.ShapeDtypeStruct((n, l), jnp.float32),
                   jax.ShapeDtypeStruct((n, l), jnp.float32)],
    )(acc0, acc1, rcol, bias, w1, w2)
    return u, v


def _final_body(acc0_ref, acc1_ref, r_ref, row_ref, out_ref):
    acc = acc0_ref[...] + acc1_ref[...]
    out_ref[...] = acc * r_ref[...] + row_ref[...]


def _final(acc0, acc1, rcol, row, rb=1000):
    n, d = acc0.shape
    nb = n // rb
    return pl.pallas_call(
        _final_body,
        grid=(nb,),
        in_specs=[pl.BlockSpec((rb, d), lambda i: (i, 0)),
                  pl.BlockSpec((rb, d), lambda i: (i, 0)),
                  pl.BlockSpec((rb, 1), lambda i: (i, 0)),
                  pl.BlockSpec((1, d), lambda i: (0, 0))],
        out_specs=pl.BlockSpec((rb, d), lambda i: (i, 0)),
        out_shape=jax.ShapeDtypeStruct((n, d), jnp.float32),
    )(acc0, acc1, rcol, row)


def _sload(ref1d, i):
    return ref1d[pl.ds(i, LANES)][0]


def _make_edge_kernel(has_edge, e_total, n_nodes, n_pad, d):
    epw = e_total // NW
    nchunks = epw // EB
    assert e_total % NW == 0 and epw % EB == 0 and EB % LANES == 0
    rows_per_tile = n_pad // NS
    assert n_pad % NS == 0 and rows_per_tile % EB == 0
    nzc = rows_per_tile // EB
    groups = EB // LANES
    nch = d // LANES

    def body(*refs):
        n_in = 8 if has_edge else 5
        n_out = 4 if has_edge else 3
        ins, outs, scr = refs[:n_in], refs[n_in:n_in + n_out], refs[n_in + n_out:]
        if has_edge:
            (src_hbm, dst_hbm, u_hbm, v_hbm, att_hbm, wev_hbm, pprev_hbm,
             rprev_hbm) = ins
            p_out, sacc_out, acc_out, alpha_out = outs
            (u_buf, v_buf, src_idx, dst_idx, dst_pad, p_buf, att_buf, s_tile,
             acc_sh, sem, wev_buf, pprev_buf, alpha_buf, rprev_buf) = scr
        else:
            src_hbm, dst_hbm, u_hbm, v_hbm, att_hbm = ins
            p_out, sacc_out, acc_out = outs
            (u_buf, v_buf, src_idx, dst_idx, dst_pad, p_buf, att_buf, s_tile,
             acc_sh, sem) = scr

        c = lax.axis_index("c")
        sx = lax.axis_index("s")
        wid = c * NS + sx
        tile_base = wid * epw
        row0 = sx * rows_per_tile

        pltpu.sync_copy(att_hbm, att_buf)
        if has_edge:
            pltpu.sync_copy(wev_hbm, wev_buf)
            pltpu.sync_copy(rprev_hbm, rprev_buf.at[pl.ds(0, n_nodes)])

        def _zu(i, _):
            for j in range(nch):
                u_buf[i, pl.ds(j * LANES, LANES)] = jnp.zeros((LANES,),
                                                              jnp.float32)
            return 0
        lax.fori_loop(0, EB, _zu, 0)

        def _zs(i, _):
            s_tile[pl.ds(i * LANES, LANES)] = jnp.zeros((LANES,), jnp.float32)
            return 0
        lax.fori_loop(0, (n_pad + LANES) // LANES, _zs, 0)

        for j in range(nzc):
            pltpu.sync_copy(u_buf, acc_sh.at[pl.ds(row0 + j * EB, EB)])
        plsc.subcore_barrier()

        def chunk(ci, _):
            base = tile_base + ci * EB
            dmas = [pltpu.async_copy(src_hbm.at[pl.ds(base, EB)], src_idx,
                                     sem),
                    pltpu.async_copy(dst_hbm.at[pl.ds(base, EB)], dst_idx,
                                     sem),
                    pltpu.async_copy(dst_hbm.at[pl.ds(base, EB)],
                                     dst_pad.at[pl.ds(0, EB)], sem)]
            if has_edge:
                dmas.append(pltpu.async_copy(pprev_hbm.at[pl.ds(base, EB)],
                                             pprev_buf.at[pl.ds(0, EB)], sem))
            for dd in dmas:
                dd.wait()
            gs = [pltpu.async_copy(u_hbm.at[src_idx], u_buf, sem),
                  pltpu.async_copy(v_hbm.at[dst_idx], v_buf, sem)]
            for dd in gs:
                dd.wait()

            att_c = [att_buf[pl.ds(j * LANES, LANES)] for j in range(nch)]
            if has_edge:
                wev_c = [wev_buf[pl.ds(j * LANES, LANES)] for j in range(nch)]

            for g in range(groups):
                lane = lax.iota(jnp.int32, LANES)

                def edgeA(l, carry):
                    lv, av = carry
                    e = g * LANES + l
                    if has_edge:
                        alpha = _sload(pprev_buf, e) * \
                            _sload(rprev_buf, _sload(dst_pad, e))
                        av = jnp.where(lane == l, alpha, av)
                    acc = jnp.zeros((LANES,), jnp.float32)
                    for j in range(nch):
                        z = u_buf[e, pl.ds(j * LANES, LANES)] + \
                            v_buf[e, pl.ds(j * LANES, LANES)]
                        if has_edge:
                            z = z + alpha * wev_c[j]
                        acc = acc + att_c[j] * jnp.maximum(z, 0.2 * z)
                    lv = jnp.where(lane == l, jnp.sum(acc), lv)
                    return (lv, av)
                z16 = jnp.zeros((LANES,), jnp.float32)
                lv, av = lax.fori_loop(0, LANES, edgeA, (z16, z16))
                p_buf[pl.ds(g * LANES, LANES)] = jnp.exp(lv)
                if has_edge:
                    alpha_buf[pl.ds(g * LANES, LANES)] = av

            def edgeB(e, _):
                p = _sload(p_buf, e)
                dste = _sload(dst_pad, e)
                win = s_tile[pl.ds(dste, LANES)]
                s_tile[pl.ds(dste, LANES)] = win + jnp.where(
                    lax.iota(jnp.int32, LANES) == 0, p, 0.0)
                for j in range(nch):
                    u_buf[e, pl.ds(j * LANES, LANES)] = \
                        p * u_buf[e, pl.ds(j * LANES, LANES)]
                return 0
            lax.fori_loop(0, EB, edgeB, 0)

            pltpu.sync_copy(p_buf.at[pl.ds(0, EB)], p_out.at[pl.ds(base, EB)])
            if has_edge:
                pltpu.sync_copy(alpha_buf, alpha_out.at[pl.ds(base, EB)])
            pltpu.sync_copy(u_buf, acc_sh.at[dst_idx], add=True)
            return 0

        lax.fori_loop(0, nchunks, chunk, 0)
        plsc.subcore_barrier()

        out_base = c * n_pad + row0
        for j in range(nzc):
            pltpu.sync_copy(acc_sh.at[pl.ds(row0 + j * EB, EB)], u_buf)
            pltpu.sync_copy(u_buf, acc_out.at[pl.ds(out_base + j * EB, EB)])
        pltpu.sync_copy(s_tile.at[pl.ds(0, n_pad)],
                        sacc_out.at[pl.ds(wid * n_pad, n_pad)])

    out_type = [jax.ShapeDtypeStruct((e_total,), jnp.float32),
                jax.ShapeDtypeStruct((NW * n_pad,), jnp.float32),
                jax.ShapeDtypeStruct((NC * n_pad, d), jnp.float32)]
    scratch = [pltpu.VMEM((EB, d), jnp.float32),
               pltpu.VMEM((EB, d), jnp.float32),
               pltpu.VMEM((EB,), jnp.int32),
               pltpu.VMEM((EB,), jnp.int32),
               pltpu.VMEM((EB + LANES,), jnp.int32),
               pltpu.VMEM((EB + LANES,), jnp.float32),
               pltpu.VMEM((d,), jnp.float32),
               pltpu.VMEM((n_pad + LANES,), jnp.float32),
               pltpu.VMEM_SHARED((n_pad, d), jnp.float32),
               pltpu.SemaphoreType.DMA]
    if has_edge:
        out_type.append(jax.ShapeDtypeStruct((e_total,), jnp.float32))
        scratch += [pltpu.VMEM((d,), jnp.float32),
                    pltpu.VMEM((EB + LANES,), jnp.float32),
                    pltpu.VMEM((EB,), jnp.float32),
                    pltpu.VMEM((n_nodes + LANES,), jnp.float32)]

    mesh = plsc.VectorSubcoreMesh(core_axis_name="c", subcore_axis_name="s")
    return pl.kernel(body, out_type=out_type, mesh=mesh, scratch_types=scratch,
                     compiler_params=pltpu.CompilerParams(
                         needs_layout_passes=False))


def kernel(x, edge_index, mmse_score, W_src1, W_dst1, att1, bias1,
           W_src2, W_dst2, att2, W_edge2, bias2, W_mmse, b_mmse):
    n, d = x.shape
    e_total = edge_index.shape[1]
    src = edge_index[0]
    dst = edge_index[1]

    n_pad = 10240 if n == 10000 else ((n + NS * EB - 1) // (NS * EB)) * (NS * EB)

    u1, v1 = _dual_matmul(x, W_src1, W_dst1)

    layer1 = _make_edge_kernel(False, e_total, n, n_pad, d)
    p1, sacc1, acc1 = layer1(src, dst, u1, v1, att1)

    r1 = _sreduce(sacc1.reshape(NW, n_pad))
    u2, v2 = _norm_mm2(acc1[:n], acc1[n_pad:n_pad + n], r1[:n],
                       bias1.reshape(1, d), W_src2, W_dst2)

    layer2 = _make_edge_kernel(True, e_total, n, n_pad, d)
    p2, sacc2, acc2, alpha1 = layer2(src, dst, u2, v2, att2, W_edge2[0],
                                     p1, r1.reshape(n_pad)[:n])

    r2 = _sreduce(sacc2.reshape(NW, n_pad))
    row = mmse_score[0, 0] * W_mmse + b_mmse[None, :] + bias2[None, :]
    gf = _final(acc2[:n], acc2[n_pad:n_pad + n], r2[:n], row.reshape(1, d))
    return (gf, alpha1)


# speedup vs baseline: 1.4689x; 1.1004x over previous
import jax
import jax.numpy as jnp
from jax import lax
from jax.experimental import pallas as pl
from jax.experimental.pallas import tpu as pltpu
from jax.experimental.pallas import tpu_sc as plsc

LANES = 16
NC = 2
NS = 16
NW = NC * NS
EB = 80


def _mm2_body(x_ref, w1_ref, w2_ref, u_ref, v_ref):
    xb = x_ref[...]
    u_ref[...] = jnp.dot(xb, w1_ref[...], preferred_element_type=jnp.float32)
    v_ref[...] = jnp.dot(xb, w2_ref[...], preferred_element_type=jnp.float32)


def _dual_matmul(x, w1, w2, rb=1000):
    n, d = x.shape
    l = w1.shape[1]
    return pl.pallas_call(
        _mm2_body,
        grid=(n // rb,),
        in_specs=[pl.BlockSpec((rb, d), lambda i: (i, 0)),
                  pl.BlockSpec((d, l), lambda i: (0, 0)),
                  pl.BlockSpec((d, l), lambda i: (0, 0))],
        out_specs=[pl.BlockSpec((rb, l), lambda i: (i, 0)),
                   pl.BlockSpec((rb, l), lambda i: (i, 0))],
        out_shape=[jax.ShapeDtypeStruct((n, l), jnp.float32),
                   jax.ShapeDtypeStruct((n, l), jnp.float32)],
    )(x, w1, w2)


def _sreduce_body(s_ref, r_ref):
    s = jnp.sum(s_ref[...], axis=0)
    r_ref[...] = (1.0 / (s + 1e-16))[:, None]


def _sreduce(s_mat):
    nw, np_ = s_mat.shape
    return pl.pallas_call(
        _sreduce_body,
        grid=(1,),
        in_specs=[pl.BlockSpec((nw, np_), lambda i: (0, 0))],
        out_specs=pl.BlockSpec((np_, 1), lambda i: (0, 0)),
        out_shape=jax.ShapeDtypeStruct((np_, 1), jnp.float32),
    )(s_mat)


def _norm_mm2_body(acc0_ref, acc1_ref, r_ref, bias_ref, w1_ref, w2_ref,
                   u_ref, v_ref):
    acc = acc0_ref[...] + acc1_ref[...]
    o = acc * r_ref[...] + bias_ref[...]
    u_ref[...] = jnp.dot(o, w1_ref[...], preferred_element_type=jnp.float32)
    v_ref[...] = jnp.dot(o, w2_ref[...], preferred_element_type=jnp.float32)


def _norm_mm2(acc0, acc1, rcol, bias, w1, w2, rb=1000):
    n, d = acc0.shape
    l = w1.shape[1]
    nb = n // rb
    u, v = pl.pallas_call(
        _norm_mm2_body,
        grid=(nb,),
        in_specs=[pl.BlockSpec((rb, d), lambda i: (i, 0)),
                  pl.BlockSpec((rb, d), lambda i: (i, 0)),
                  pl.BlockSpec((rb, 1), lambda i: (i, 0)),
                  pl.BlockSpec((1, d), lambda i: (0, 0)),
                  pl.BlockSpec((d, l), lambda i: (0, 0)),
                  pl.BlockSpec((d, l), lambda i: (0, 0))],
        out_specs=[pl.BlockSpec((rb, l), lambda i: (i, 0)),
                   pl.BlockSpec((rb, l), lambda i: (i, 0))],
        out_shape=[jax.ShapeDtypeStruct((n, l), jnp.float32),
                   jax.ShapeDtypeStruct((n, l), jnp.float32)],
    )(acc0, acc1, rcol, bias, w1, w2)
    return u, v


def _final_body(acc0_ref, acc1_ref, r_ref, row_ref, out_ref):
    acc = acc0_ref[...] + acc1_ref[...]
    out_ref[...] = acc * r_ref[...] + row_ref[...]


def _final(acc0, acc1, rcol, row, rb=1000):
    n, d = acc0.shape
    nb = n // rb
    return pl.pallas_call(
        _final_body,
        grid=(nb,),
        in_specs=[pl.BlockSpec((rb, d), lambda i: (i, 0)),
                  pl.BlockSpec((rb, d), lambda i: (i, 0)),
                  pl.BlockSpec((rb, 1), lambda i: (i, 0)),
                  pl.BlockSpec((1, d), lambda i: (0, 0))],
        out_specs=pl.BlockSpec((rb, d), lambda i: (i, 0)),
        out_shape=jax.ShapeDtypeStruct((n, d), jnp.float32),
    )(acc0, acc1, rcol, row)


def _sload(ref1d, i):
    return ref1d[pl.ds(i, LANES)][0]


def _make_edge_kernel(has_edge, e_total, n_nodes, n_pad, d):
    epw = e_total // NW
    nchunks = epw // EB
    assert e_total % NW == 0 and epw % EB == 0 and EB % LANES == 0
    rows_per_tile = n_pad // NS
    assert n_pad % NS == 0 and rows_per_tile % EB == 0
    nzc = rows_per_tile // EB
    groups = EB // LANES
    nch = d // LANES

    def body(*refs):
        n_in = 8 if has_edge else 5
        n_out = 4 if has_edge else 3
        ins, outs, scr = refs[:n_in], refs[n_in:n_in + n_out], refs[n_in + n_out:]
        if has_edge:
            (src_hbm, dst_hbm, u_hbm, v_hbm, att_hbm, wev_hbm, pprev_hbm,
             rprev_hbm) = ins
            p_out, sacc_out, acc_out, alpha_out = outs
            (u_buf, v_buf, src_idx, dst_idx, dst_pad, p_buf, att_buf, s_tile,
             acc_sh, sem, wev_buf, pprev_buf, alpha_buf, rprev_buf) = scr
        else:
            src_hbm, dst_hbm, u_hbm, v_hbm, att_hbm = ins
            p_out, sacc_out, acc_out = outs
            (u_buf, v_buf, src_idx, dst_idx, dst_pad, p_buf, att_buf, s_tile,
             acc_sh, sem) = scr

        c = lax.axis_index("c")
        sx = lax.axis_index("s")
        wid = c * NS + sx
        tile_base = wid * epw
        row0 = sx * rows_per_tile

        pltpu.sync_copy(att_hbm, att_buf)
        if has_edge:
            pltpu.sync_copy(wev_hbm, wev_buf)
            pltpu.sync_copy(rprev_hbm, rprev_buf.at[pl.ds(0, n_nodes)])

        def _zu(i, _):
            for j in range(nch):
                u_buf[i, pl.ds(j * LANES, LANES)] = jnp.zeros((LANES,),
                                                              jnp.float32)
            return 0
        lax.fori_loop(0, EB, _zu, 0)

        def _zs(i, _):
            s_tile[pl.ds(i * LANES, LANES)] = jnp.zeros((LANES,), jnp.float32)
            return 0
        lax.fori_loop(0, (n_pad + LANES) // LANES, _zs, 0)

        for j in range(nzc):
            pltpu.sync_copy(u_buf, acc_sh.at[pl.ds(row0 + j * EB, EB)])
        plsc.subcore_barrier()

        def chunk(ci, _):
            base = tile_base + ci * EB
            dmas = [pltpu.async_copy(src_hbm.at[pl.ds(base, EB)], src_idx,
                                     sem),
                    pltpu.async_copy(dst_hbm.at[pl.ds(base, EB)], dst_idx,
                                     sem),
                    pltpu.async_copy(dst_hbm.at[pl.ds(base, EB)],
                                     dst_pad.at[pl.ds(0, EB)], sem)]
            if has_edge:
                dmas.append(pltpu.async_copy(pprev_hbm.at[pl.ds(base, EB)],
                                             pprev_buf.at[pl.ds(0, EB)], sem))
            for dd in dmas:
                dd.wait()
            h0 = 2 * LANES
            h1 = EB - h0
            g0 = [pltpu.async_copy(u_hbm.at[src_idx.at[pl.ds(0, h0)]],
                                   u_buf.at[pl.ds(0, h0)], sem),
                  pltpu.async_copy(v_hbm.at[dst_idx.at[pl.ds(0, h0)]],
                                   v_buf.at[pl.ds(0, h0)], sem)]
            g1 = [pltpu.async_copy(u_hbm.at[src_idx.at[pl.ds(h0, h1)]],
                                   u_buf.at[pl.ds(h0, h1)], sem),
                  pltpu.async_copy(v_hbm.at[dst_idx.at[pl.ds(h0, h1)]],
                                   v_buf.at[pl.ds(h0, h1)], sem)]
            for dd in g0:
                dd.wait()

            att_c = [att_buf[pl.ds(j * LANES, LANES)] for j in range(nch)]
            if has_edge:
                wev_c = [wev_buf[pl.ds(j * LANES, LANES)] for j in range(nch)]

            def passA(g):
                lane = lax.iota(jnp.int32, LANES)

                def edgeA(l, carry):
                    lv, av = carry
                    e = g * LANES + l
                    if has_edge:
                        alpha = _sload(pprev_buf, e) * \
                            _sload(rprev_buf, _sload(dst_pad, e))
                        av = jnp.where(lane == l, alpha, av)
                    acc = jnp.zeros((LANES,), jnp.float32)
                    for j in range(nch):
                        z = u_buf[e, pl.ds(j * LANES, LANES)] + \
                            v_buf[e, pl.ds(j * LANES, LANES)]
                        if has_edge:
                            z = z + alpha * wev_c[j]
                        acc = acc + att_c[j] * jnp.maximum(z, 0.2 * z)
                    lv = jnp.where(lane == l, jnp.sum(acc), lv)
                    return (lv, av)
                z16 = jnp.zeros((LANES,), jnp.float32)
                lv, av = lax.fori_loop(0, LANES, edgeA, (z16, z16))
                p_buf[pl.ds(g * LANES, LANES)] = jnp.exp(lv)
                if has_edge:
                    alpha_buf[pl.ds(g * LANES, LANES)] = av

            def edgeB(e, _):
                p = _sload(p_buf, e)
                dste = _sload(dst_pad, e)
                win = s_tile[pl.ds(dste, LANES)]
                s_tile[pl.ds(dste, LANES)] = win + jnp.where(
                    lax.iota(jnp.int32, LANES) == 0, p, 0.0)
                for j in range(nch):
                    u_buf[e, pl.ds(j * LANES, LANES)] = \
                        p * u_buf[e, pl.ds(j * LANES, LANES)]
                return 0

            for g in range(2):
                passA(g)
            lax.fori_loop(0, h0, edgeB, 0)
            for dd in g1:
                dd.wait()
            for g in range(2, groups):
                passA(g)
            lax.fori_loop(h0, EB, edgeB, 0)

            outs_d = [pltpu.async_copy(p_buf.at[pl.ds(0, EB)],
                                       p_out.at[pl.ds(base, EB)], sem)]
            if has_edge:
                outs_d.append(pltpu.async_copy(alpha_buf,
                                               alpha_out.at[pl.ds(base, EB)],
                                               sem))
            pltpu.sync_copy(u_buf, acc_sh.at[dst_idx], add=True)
            for dd in outs_d:
                dd.wait()
            return 0

        lax.fori_loop(0, nchunks, chunk, 0)
        plsc.subcore_barrier()

        out_base = c * n_pad + row0
        for j in range(nzc):
            pltpu.sync_copy(acc_sh.at[pl.ds(row0 + j * EB, EB)], u_buf)
            pltpu.sync_copy(u_buf, acc_out.at[pl.ds(out_base + j * EB, EB)])
        pltpu.sync_copy(s_tile.at[pl.ds(0, n_pad)],
                        sacc_out.at[pl.ds(wid * n_pad, n_pad)])

    out_type = [jax.ShapeDtypeStruct((e_total,), jnp.float32),
                jax.ShapeDtypeStruct((NW * n_pad,), jnp.float32),
                jax.ShapeDtypeStruct((NC * n_pad, d), jnp.float32)]
    scratch = [pltpu.VMEM((EB, d), jnp.float32),
               pltpu.VMEM((EB, d), jnp.float32),
               pltpu.VMEM((EB,), jnp.int32),
               pltpu.VMEM((EB,), jnp.int32),
               pltpu.VMEM((EB + LANES,), jnp.int32),
               pltpu.VMEM((EB + LANES,), jnp.float32),
               pltpu.VMEM((d,), jnp.float32),
               pltpu.VMEM((n_pad + LANES,), jnp.float32),
               pltpu.VMEM_SHARED((n_pad, d), jnp.float32),
               pltpu.SemaphoreType.DMA]
    if has_edge:
        out_type.append(jax.ShapeDtypeStruct((e_total,), jnp.float32))
        scratch += [pltpu.VMEM((d,), jnp.float32),
                    pltpu.VMEM((EB + LANES,), jnp.float32),
                    pltpu.VMEM((EB,), jnp.float32),
                    pltpu.VMEM((n_nodes + LANES,), jnp.float32)]

    mesh = plsc.VectorSubcoreMesh(core_axis_name="c", subcore_axis_name="s")
    return pl.kernel(body, out_type=out_type, mesh=mesh, scratch_types=scratch,
                     compiler_params=pltpu.CompilerParams(
                         needs_layout_passes=False))


def kernel(x, edge_index, mmse_score, W_src1, W_dst1, att1, bias1,
           W_src2, W_dst2, att2, W_edge2, bias2, W_mmse, b_mmse):
    n, d = x.shape
    e_total = edge_index.shape[1]
    src = edge_index[0]
    dst = edge_index[1]

    n_pad = 10240 if n == 10000 else ((n + NS * EB - 1) // (NS * EB)) * (NS * EB)

    u1, v1 = _dual_matmul(x, W_src1, W_dst1)

    layer1 = _make_edge_kernel(False, e_total, n, n_pad, d)
    p1, sacc1, acc1 = layer1(src, dst, u1, v1, att1)

    r1 = _sreduce(sacc1.reshape(NW, n_pad))
    u2, v2 = _norm_mm2(acc1[:n], acc1[n_pad:n_pad + n], r1[:n],
                       bias1.reshape(1, d), W_src2, W_dst2)

    layer2 = _make_edge_kernel(True, e_total, n, n_pad, d)
    p2, sacc2, acc2, alpha1 = layer2(src, dst, u2, v2, att2, W_edge2[0],
                                     p1, r1.reshape(n_pad)[:n])

    r2 = _sreduce(sacc2.reshape(NW, n_pad))
    row = mmse_score[0, 0] * W_mmse + b_mmse[None, :] + bias2[None, :]
    gf = _final(acc2[:n], acc2[n_pad:n_pad + n], r2[:n], row.reshape(1, d))
    return (gf, alpha1)


# speedup vs baseline: 1.4731x; 1.0029x over previous
import jax
import jax.numpy as jnp
from jax import lax
from jax.experimental import pallas as pl
from jax.experimental.pallas import tpu as pltpu
from jax.experimental.pallas import tpu_sc as plsc

LANES = 16
NC = 2
NS = 16
NW = NC * NS
EB = 80


def _mm2_body(x_ref, w1_ref, w2_ref, u_ref, v_ref):
    xb = x_ref[...]
    u_ref[...] = jnp.dot(xb, w1_ref[...], preferred_element_type=jnp.float32)
    v_ref[...] = jnp.dot(xb, w2_ref[...], preferred_element_type=jnp.float32)


def _dual_matmul(x, w1, w2, rb=1000):
    n, d = x.shape
    l = w1.shape[1]
    return pl.pallas_call(
        _mm2_body,
        grid=(n // rb,),
        in_specs=[pl.BlockSpec((rb, d), lambda i: (i, 0)),
                  pl.BlockSpec((d, l), lambda i: (0, 0)),
                  pl.BlockSpec((d, l), lambda i: (0, 0))],
        out_specs=[pl.BlockSpec((rb, l), lambda i: (i, 0)),
                   pl.BlockSpec((rb, l), lambda i: (i, 0))],
        out_shape=[jax.ShapeDtypeStruct((n, l), jnp.float32),
                   jax.ShapeDtypeStruct((n, l), jnp.float32)],
    )(x, w1, w2)


def _sreduce_body(s_ref, r_ref):
    s = jnp.sum(s_ref[...], axis=0)
    r_ref[...] = (1.0 / (s + 1e-16))[:, None]


def _sreduce(s_mat):
    nw, np_ = s_mat.shape
    return pl.pallas_call(
        _sreduce_body,
        grid=(1,),
        in_specs=[pl.BlockSpec((nw, np_), lambda i: (0, 0))],
        out_specs=pl.BlockSpec((np_, 1), lambda i: (0, 0)),
        out_shape=jax.ShapeDtypeStruct((np_, 1), jnp.float32),
    )(s_mat)


def _norm_mm2_body(acc0_ref, acc1_ref, r_ref, bias_ref, w1_ref, w2_ref,
                   u_ref, v_ref):
    acc = acc0_ref[...] + acc1_ref[...]
    o = acc * r_ref[...] + bias_ref[...]
    u_ref[...] = jnp.dot(o, w1_ref[...], preferred_element_type=jnp.float32)
    v_ref[...] = jnp.dot(o, w2_ref[...], preferred_element_type=jnp.float32)


def _norm_mm2(acc0, acc1, rcol, bias, w1, w2, rb=1000):
    n, d = acc0.shape
    l = w1.shape[1]
    nb = n // rb
    u, v = pl.pallas_call(
        _norm_mm2_body,
        grid=(nb,),
        in_specs=[pl.BlockSpec((rb, d), lambda i: (i, 0)),
                  pl.BlockSpec((rb, d), lambda i: (i, 0)),
                  pl.BlockSpec((rb, 1), lambda i: (i, 0)),
                  pl.BlockSpec((1, d), lambda i: (0, 0)),
                  pl.BlockSpec((d, l), lambda i: (0, 0)),
                  pl.BlockSpec((d, l), lambda i: (0, 0))],
        out_specs=[pl.BlockSpec((rb, l), lambda i: (i, 0)),
                   pl.BlockSpec((rb, l), lambda i: (i, 0))],
        out_shape=[jax.ShapeDtypeStruct((n, l), jnp.float32),
                   jax.ShapeDtypeStruct((n, l), jnp.float32)],
    )(acc0, acc1, rcol, bias, w1, w2)
    return u, v


def _final_body(acc0_ref, acc1_ref, r_ref, row_ref, out_ref):
    acc = acc0_ref[...] + acc1_ref[...]
    out_ref[...] = acc * r_ref[...] + row_ref[...]


def _final(acc0, acc1, rcol, row, rb=1000):
    n, d = acc0.shape
    nb = n // rb
    return pl.pallas_call(
        _final_body,
        grid=(nb,),
        in_specs=[pl.BlockSpec((rb, d), lambda i: (i, 0)),
                  pl.BlockSpec((rb, d), lambda i: (i, 0)),
                  pl.BlockSpec((rb, 1), lambda i: (i, 0)),
                  pl.BlockSpec((1, d), lambda i: (0, 0))],
        out_specs=pl.BlockSpec((rb, d), lambda i: (i, 0)),
        out_shape=jax.ShapeDtypeStruct((n, d), jnp.float32),
    )(acc0, acc1, rcol, row)


def _sload(ref1d, i):
    return ref1d[pl.ds(i, LANES)][0]


def _make_edge_kernel(has_edge, e_total, n_nodes, n_pad, d):
    epw = e_total // NW
    nchunks = epw // EB
    assert e_total % NW == 0 and epw % EB == 0 and EB % LANES == 0
    rows_per_tile = n_pad // NS
    assert n_pad % NS == 0 and rows_per_tile % EB == 0
    nzc = rows_per_tile // EB
    groups = EB // LANES
    nch = d // LANES

    def body(*refs):
        n_in = 8 if has_edge else 5
        n_out = 4 if has_edge else 3
        ins, outs, scr = refs[:n_in], refs[n_in:n_in + n_out], refs[n_in + n_out:]
        if has_edge:
            (src_hbm, dst_hbm, u_hbm, v_hbm, att_hbm, wev_hbm, pprev_hbm,
             rprev_hbm) = ins
            p_out, sacc_out, acc_out, alpha_out = outs
            (u_buf, v_buf, src_idx, dst_idx, dst_pad, p_buf, att_buf, s_tile,
             acc_sh, sem, wev_buf, pprev_buf, alpha_buf, rprev_buf) = scr
        else:
            src_hbm, dst_hbm, u_hbm, v_hbm, att_hbm = ins
            p_out, sacc_out, acc_out = outs
            (u_buf, v_buf, src_idx, dst_idx, dst_pad, p_buf, att_buf, s_tile,
             acc_sh, sem) = scr

        c = lax.axis_index("c")
        sx = lax.axis_index("s")
        wid = c * NS + sx
        tile_base = wid * epw
        row0 = sx * rows_per_tile

        pltpu.sync_copy(att_hbm, att_buf)
        if has_edge:
            pltpu.sync_copy(wev_hbm, wev_buf)
            pltpu.sync_copy(rprev_hbm, rprev_buf.at[pl.ds(0, n_nodes)])

        def _zu(i, _):
            for j in range(nch):
                u_buf[i, pl.ds(j * LANES, LANES)] = jnp.zeros((LANES,),
                                                              jnp.float32)
            return 0
        lax.fori_loop(0, EB, _zu, 0)

        def _zs(i, _):
            s_tile[pl.ds(i * LANES, LANES)] = jnp.zeros((LANES,), jnp.float32)
            return 0
        lax.fori_loop(0, (n_pad + LANES) // LANES, _zs, 0)

        for j in range(nzc):
            pltpu.sync_copy(u_buf, acc_sh.at[pl.ds(row0 + j * EB, EB)])
        plsc.subcore_barrier()

        def chunk(ci, _):
            base = tile_base + ci * EB
            dmas = [pltpu.async_copy(src_hbm.at[pl.ds(base, EB)], src_idx,
                                     sem),
                    pltpu.async_copy(dst_hbm.at[pl.ds(base, EB)], dst_idx,
                                     sem),
                    pltpu.async_copy(dst_hbm.at[pl.ds(base, EB)],
                                     dst_pad.at[pl.ds(0, EB)], sem)]
            if has_edge:
                dmas.append(pltpu.async_copy(pprev_hbm.at[pl.ds(base, EB)],
                                             pprev_buf.at[pl.ds(0, EB)], sem))
            for dd in dmas:
                dd.wait()
            h0 = 2 * LANES
            h1 = EB - h0
            g0 = [pltpu.async_copy(u_hbm.at[src_idx.at[pl.ds(0, h0)]],
                                   u_buf.at[pl.ds(0, h0)], sem),
                  pltpu.async_copy(v_hbm.at[dst_idx.at[pl.ds(0, h0)]],
                                   v_buf.at[pl.ds(0, h0)], sem)]
            g1 = [pltpu.async_copy(u_hbm.at[src_idx.at[pl.ds(h0, h1)]],
                                   u_buf.at[pl.ds(h0, h1)], sem),
                  pltpu.async_copy(v_hbm.at[dst_idx.at[pl.ds(h0, h1)]],
                                   v_buf.at[pl.ds(h0, h1)], sem)]
            for dd in g0:
                dd.wait()

            att_c = [att_buf[pl.ds(j * LANES, LANES)] for j in range(nch)]
            if has_edge:
                wev_c = [wev_buf[pl.ds(j * LANES, LANES)] for j in range(nch)]

            def passA(g):
                lane = lax.iota(jnp.int32, LANES)

                def edgeA(l, carry):
                    lv, av = carry
                    e = g * LANES + l
                    if has_edge:
                        alpha = _sload(pprev_buf, e) * \
                            _sload(rprev_buf, _sload(dst_pad, e))
                        av = jnp.where(lane == l, alpha, av)
                    acc = jnp.zeros((LANES,), jnp.float32)
                    for j in range(nch):
                        z = u_buf[e, pl.ds(j * LANES, LANES)] + \
                            v_buf[e, pl.ds(j * LANES, LANES)]
                        if has_edge:
                            z = z + alpha * wev_c[j]
                        acc = acc + att_c[j] * jnp.maximum(z, 0.2 * z)
                    lv = jnp.where(lane == l, jnp.sum(acc), lv)
                    return (lv, av)
                z16 = jnp.zeros((LANES,), jnp.float32)
                lv, av = lax.fori_loop(0, LANES, edgeA, (z16, z16),
                                       unroll=2)
                p_buf[pl.ds(g * LANES, LANES)] = jnp.exp(lv)
                if has_edge:
                    alpha_buf[pl.ds(g * LANES, LANES)] = av

            def edgeB(e, _):
                p = _sload(p_buf, e)
                dste = _sload(dst_pad, e)
                win = s_tile[pl.ds(dste, LANES)]
                s_tile[pl.ds(dste, LANES)] = win + jnp.where(
                    lax.iota(jnp.int32, LANES) == 0, p, 0.0)
                for j in range(nch):
                    u_buf[e, pl.ds(j * LANES, LANES)] = \
                        p * u_buf[e, pl.ds(j * LANES, LANES)]
                return 0

            for g in range(2):
                passA(g)
            lax.fori_loop(0, h0, edgeB, 0)
            for dd in g1:
                dd.wait()
            for g in range(2, groups):
                passA(g)
            lax.fori_loop(h0, EB, edgeB, 0)

            outs_d = [pltpu.async_copy(p_buf.at[pl.ds(0, EB)],
                                       p_out.at[pl.ds(base, EB)], sem)]
            if has_edge:
                outs_d.append(pltpu.async_copy(alpha_buf,
                                               alpha_out.at[pl.ds(base, EB)],
                                               sem))
            pltpu.sync_copy(u_buf, acc_sh.at[dst_idx], add=True)
            for dd in outs_d:
                dd.wait()
            return 0

        lax.fori_loop(0, nchunks, chunk, 0)
        plsc.subcore_barrier()

        out_base = c * n_pad + row0
        for j in range(nzc):
            pltpu.sync_copy(acc_sh.at[pl.ds(row0 + j * EB, EB)], u_buf)
            pltpu.sync_copy(u_buf, acc_out.at[pl.ds(out_base + j * EB, EB)])
        pltpu.sync_copy(s_tile.at[pl.ds(0, n_pad)],
                        sacc_out.at[pl.ds(wid * n_pad, n_pad)])

    out_type = [jax.ShapeDtypeStruct((e_total,), jnp.float32),
                jax.ShapeDtypeStruct((NW * n_pad,), jnp.float32),
                jax.ShapeDtypeStruct((NC * n_pad, d), jnp.float32)]
    scratch = [pltpu.VMEM((EB, d), jnp.float32),
               pltpu.VMEM((EB, d), jnp.float32),
               pltpu.VMEM((EB,), jnp.int32),
               pltpu.VMEM((EB,), jnp.int32),
               pltpu.VMEM((EB + LANES,), jnp.int32),
               pltpu.VMEM((EB + LANES,), jnp.float32),
               pltpu.VMEM((d,), jnp.float32),
               pltpu.VMEM((n_pad + LANES,), jnp.float32),
               pltpu.VMEM_SHARED((n_pad, d), jnp.float32),
               pltpu.SemaphoreType.DMA]
    if has_edge:
        out_type.append(jax.ShapeDtypeStruct((e_total,), jnp.float32))
        scratch += [pltpu.VMEM((d,), jnp.float32),
                    pltpu.VMEM((EB + LANES,), jnp.float32),
                    pltpu.VMEM((EB,), jnp.float32),
                    pltpu.VMEM((n_nodes + LANES,), jnp.float32)]

    mesh = plsc.VectorSubcoreMesh(core_axis_name="c", subcore_axis_name="s")
    return pl.kernel(body, out_type=out_type, mesh=mesh, scratch_types=scratch,
                     compiler_params=pltpu.CompilerParams(
                         needs_layout_passes=False))


def kernel(x, edge_index, mmse_score, W_src1, W_dst1, att1, bias1,
           W_src2, W_dst2, att2, W_edge2, bias2, W_mmse, b_mmse):
    n, d = x.shape
    e_total = edge_index.shape[1]
    src = edge_index[0]
    dst = edge_index[1]

    n_pad = 10240 if n == 10000 else ((n + NS * EB - 1) // (NS * EB)) * (NS * EB)

    u1, v1 = _dual_matmul(x, W_src1, W_dst1)

    layer1 = _make_edge_kernel(False, e_total, n, n_pad, d)
    p1, sacc1, acc1 = layer1(src, dst, u1, v1, att1)

    r1 = _sreduce(sacc1.reshape(NW, n_pad))
    u2, v2 = _norm_mm2(acc1[:n], acc1[n_pad:n_pad + n], r1[:n],
                       bias1.reshape(1, d), W_src2, W_dst2)

    layer2 = _make_edge_kernel(True, e_total, n, n_pad, d)
    p2, sacc2, acc2, alpha1 = layer2(src, dst, u2, v2, att2, W_edge2[0],
                                     p1, r1.reshape(n_pad)[:n])

    r2 = _sreduce(sacc2.reshape(NW, n_pad))
    row = mmse_score[0, 0] * W_mmse + b_mmse[None, :] + bias2[None, :]
    gf = _final(acc2[:n], acc2[n_pad:n_pad + n], r2[:n], row.reshape(1, d))
    return (gf, alpha1)
